# baseline (speedup 1.0000x reference)
import functools

import jax
import jax.numpy as jnp
from jax import lax
from jax.experimental import pallas as pl
from jax.experimental.pallas import tpu as pltpu
from jax.experimental.pallas import tpu_sc as plsc

B = 16
L = 2048
D = 256
LANES = 16
VPR = D // LANES
ROWS_PER_W = 1024
CHUNK = 128
N_CHUNKS = ROWS_PER_W // CHUNK
NBUF = 3
ZROWS = CHUNK // 2
HALF = 1024


def _body(table, idxr, lens, out, mask, idx_v, buf0, buf1, buf2, zbuf,
          mask_v, len_v, sg0, sg1, sg2, so0, so1, so2, sm):
    b = lax.axis_index("s")
    half = (b + lax.axis_index("c")) % 2
    col = half * HALF

    pltpu.sync_copy(idxr.at[b, pl.ds(col, ROWS_PER_W)], idx_v)
    pltpu.sync_copy(lens, len_v.at[pl.ds(0, LANES)])

    iota = lax.broadcasted_iota(jnp.int32, (LANES,), 0)
    thresh = len_v[pl.ds(b, LANES)][0]
    v = jnp.clip(thresh - col, 0, ROWS_PER_W)

    zeros16 = jnp.zeros((LANES,), jnp.float32)
    ones16 = jnp.ones((LANES,), jnp.float32)

    bufs = [buf0, buf1, buf2]
    sgs = [sg0, sg1, sg2]
    sos = [so0, so1, so2]
    anyv = [v > c * CHUNK for c in range(N_CHUNKS)]
    fullv = [v >= (c + 1) * CHUNK for c in range(N_CHUNKS)]

    def gather_start(c):
        p = c % NBUF
        pltpu.async_copy(table.at[idx_v.at[pl.ds(c * CHUNK, CHUNK)]],
                         bufs[p], sgs[p])

    def gather_wait(c):
        p = c % NBUF
        pltpu.make_async_copy(table.at[idx_v.at[pl.ds(c * CHUNK, CHUNK)]],
                              bufs[p], sgs[p]).wait()

    def out_start(c):
        p = c % NBUF
        pltpu.async_copy(bufs[p], out.at[b, pl.ds(col + c * CHUNK, CHUNK)],
                         sos[p])

    def zout_start(c):
        p = c % NBUF
        lo = col + c * CHUNK
        pltpu.async_copy(zbuf, out.at[b, pl.ds(lo, ZROWS)], sos[p])
        pltpu.async_copy(zbuf, out.at[b, pl.ds(lo + ZROWS, ZROWS)], sos[p])

    def out_wait(c):
        p = c % NBUF
        pltpu.make_async_copy(
            bufs[p], out.at[b, pl.ds(col + c * CHUNK, CHUNK)], sos[p]).wait()

    for k in range(NBUF - 1):
        @pl.when(anyv[k])
        def _(k=k):
            gather_start(k)

    @pl.when(v <= (N_CHUNKS - 1) * CHUNK)
    def _():
        def zrow(r, carry):
            for i in range(VPR):
                zbuf[r, pl.ds(i * LANES, LANES)] = zeros16
            return carry

        lax.fori_loop(0, ZROWS, zrow, 0)

    def mrow(i, carry):
        j = iota + (col + i * LANES)
        mask_v[pl.ds(i * LANES, LANES)] = jnp.where(j < thresh, ones16,
                                                    zeros16)
        return carry

    lax.fori_loop(0, ROWS_PER_W // LANES, mrow, 0)
    pltpu.async_copy(mask_v, mask.at[b, pl.ds(col, ROWS_PER_W)], sm)

    for c in range(N_CHUNKS):
        @pl.when(anyv[c])
        def _(c=c):
            gather_wait(c)

            @pl.when(jnp.logical_not(fullv[c]))
            def _():
                buf = bufs[c % NBUF]

                def zsuf(r, carry):
                    for i in range(VPR):
                        buf[r, pl.ds(i * LANES, LANES)] = zeros16
                    return carry

                lax.fori_loop(v - c * CHUNK, CHUNK, zsuf, 0)

            out_start(c)

        @pl.when(jnp.logical_not(anyv[c]))
        def _(c=c):
            zout_start(c)

        n = c + NBUF - 1
        if n < N_CHUNKS:
            if c >= 1:
                out_wait(c - 1)

            @pl.when(anyv[n])
            def _(n=n):
                gather_start(n)

    for c in range(N_CHUNKS - NBUF, N_CHUNKS):
        out_wait(c)
    pltpu.make_async_copy(mask_v, mask.at[b, pl.ds(col, ROWS_PER_W)],
                          sm).wait()


@jax.jit
def _sampler(table, idxr, lens):
    mesh = plsc.VectorSubcoreMesh(core_axis_name="c", subcore_axis_name="s")
    f = functools.partial(
        pl.kernel,
        mesh=mesh,
        out_type=[
            jax.ShapeDtypeStruct((B, L, D), jnp.float32),
            jax.ShapeDtypeStruct((B, L), jnp.float32),
        ],
        scratch_types=[
            pltpu.VMEM((ROWS_PER_W,), jnp.int32),
            pltpu.VMEM((CHUNK, D), jnp.float32),
            pltpu.VMEM((CHUNK, D), jnp.float32),
            pltpu.VMEM((CHUNK, D), jnp.float32),
            pltpu.VMEM((ZROWS, D), jnp.float32),
            pltpu.VMEM((ROWS_PER_W,), jnp.float32),
            pltpu.VMEM((2 * LANES,), jnp.int32),
            pltpu.SemaphoreType.DMA,
            pltpu.SemaphoreType.DMA,
            pltpu.SemaphoreType.DMA,
            pltpu.SemaphoreType.DMA,
            pltpu.SemaphoreType.DMA,
            pltpu.SemaphoreType.DMA,
            pltpu.SemaphoreType.DMA,
        ],
    )(_body)
    return f(table, idxr, lens)


def kernel(item_universe, indices, lengths):
    out, mask = _sampler(item_universe, indices.astype(jnp.int32),
                         lengths.astype(jnp.int32))
    return out, mask


# speedup vs baseline: 1.0334x; 1.0334x over previous
import functools

import jax
import jax.numpy as jnp
from jax import lax
from jax.experimental import pallas as pl
from jax.experimental.pallas import tpu as pltpu
from jax.experimental.pallas import tpu_sc as plsc

B = 16
L = 2048
D = 256
LANES = 16
VPR = D // LANES
ROWS_PER_W = 1024
CHUNK = 128
N_CHUNKS = ROWS_PER_W // CHUNK
NBUF = 3
ZROWS = CHUNK // 2
HALF = 1024


def _body(table, idxr, lens, out, mask, idx_v, buf0, buf1, buf2, zbuf,
          mask_v, len_v, sg0, sg1, sg2, so0, so1, so2, sm):
    b = lax.axis_index("s")
    half = (b + lax.axis_index("c")) % 2
    col = half * HALF

    pltpu.async_copy(idxr.at[b, pl.ds(col, ROWS_PER_W)], idx_v, sg0)
    pltpu.async_copy(lens, len_v.at[pl.ds(0, LANES)], sg1)
    pltpu.make_async_copy(lens, len_v.at[pl.ds(0, LANES)], sg1).wait()

    iota = lax.broadcasted_iota(jnp.int32, (LANES,), 0)
    thresh = len_v[pl.ds(b, LANES)][0]
    v = jnp.clip(thresh - col, 0, ROWS_PER_W)

    zeros16 = jnp.zeros((LANES,), jnp.float32)
    ones16 = jnp.ones((LANES,), jnp.float32)

    bufs = [buf0, buf1, buf2]
    sgs = [sg0, sg1, sg2]
    sos = [so0, so1, so2]
    anyv = [v > c * CHUNK for c in range(N_CHUNKS)]
    fullv = [v >= (c + 1) * CHUNK for c in range(N_CHUNKS)]

    def gather_start(c):
        p = c % NBUF
        pltpu.async_copy(table.at[idx_v.at[pl.ds(c * CHUNK, CHUNK)]],
                         bufs[p], sgs[p])

    def gather_wait(c):
        p = c % NBUF
        pltpu.make_async_copy(table.at[idx_v.at[pl.ds(c * CHUNK, CHUNK)]],
                              bufs[p], sgs[p]).wait()

    def out_start(c):
        p = c % NBUF
        pltpu.async_copy(bufs[p], out.at[b, pl.ds(col + c * CHUNK, CHUNK)],
                         sos[p])

    def zout_start(c):
        p = c % NBUF
        lo = col + c * CHUNK
        pltpu.async_copy(zbuf, out.at[b, pl.ds(lo, ZROWS)], sos[p])
        pltpu.async_copy(zbuf, out.at[b, pl.ds(lo + ZROWS, ZROWS)], sos[p])

    def out_wait(c):
        p = c % NBUF
        pltpu.make_async_copy(
            bufs[p], out.at[b, pl.ds(col + c * CHUNK, CHUNK)], sos[p]).wait()

    pltpu.make_async_copy(idxr.at[b, pl.ds(col, ROWS_PER_W)], idx_v,
                          sg0).wait()
    for k in range(NBUF - 1):
        @pl.when(anyv[k])
        def _(k=k):
            gather_start(k)

    @pl.when(v <= (N_CHUNKS - 1) * CHUNK)
    def _():
        def zrow(r, carry):
            for i in range(VPR):
                zbuf[r, pl.ds(i * LANES, LANES)] = zeros16
            return carry

        lax.fori_loop(0, ZROWS, zrow, 0)

    for c in range(N_CHUNKS):
        @pl.when(anyv[c])
        def _(c=c):
            gather_wait(c)

            @pl.when(jnp.logical_not(fullv[c]))
            def _():
                buf = bufs[c % NBUF]

                def zsuf(r, carry):
                    for i in range(VPR):
                        buf[r, pl.ds(i * LANES, LANES)] = zeros16
                    return carry

                lax.fori_loop(v - c * CHUNK, CHUNK, zsuf, 0)

            out_start(c)

        @pl.when(jnp.logical_not(anyv[c]))
        def _(c=c):
            zout_start(c)

        n = c + NBUF - 1
        if n < N_CHUNKS:
            if c >= 1:
                out_wait(c - 1)

            @pl.when(anyv[n])
            def _(n=n):
                gather_start(n)

        if c == 0:
            def mrow(i, carry):
                j = iota + (col + i * LANES)
                mask_v[pl.ds(i * LANES, LANES)] = jnp.where(
                    j < thresh, ones16, zeros16)
                return carry

            lax.fori_loop(0, ROWS_PER_W // LANES, mrow, 0)
            pltpu.async_copy(mask_v, mask.at[b, pl.ds(col, ROWS_PER_W)], sm)

    for c in range(N_CHUNKS - NBUF, N_CHUNKS):
        out_wait(c)
    pltpu.make_async_copy(mask_v, mask.at[b, pl.ds(col, ROWS_PER_W)],
                          sm).wait()


@jax.jit
def _sampler(table, idxr, lens):
    mesh = plsc.VectorSubcoreMesh(core_axis_name="c", subcore_axis_name="s")
    f = functools.partial(
        pl.kernel,
        mesh=mesh,
        out_type=[
            jax.ShapeDtypeStruct((B, L, D), jnp.float32),
            jax.ShapeDtypeStruct((B, L), jnp.float32),
        ],
        scratch_types=[
            pltpu.VMEM((ROWS_PER_W,), jnp.int32),
            pltpu.VMEM((CHUNK, D), jnp.float32),
            pltpu.VMEM((CHUNK, D), jnp.float32),
            pltpu.VMEM((CHUNK, D), jnp.float32),
            pltpu.VMEM((ZROWS, D), jnp.float32),
            pltpu.VMEM((ROWS_PER_W,), jnp.float32),
            pltpu.VMEM((2 * LANES,), jnp.int32),
            pltpu.SemaphoreType.DMA,
            pltpu.SemaphoreType.DMA,
            pltpu.SemaphoreType.DMA,
            pltpu.SemaphoreType.DMA,
            pltpu.SemaphoreType.DMA,
            pltpu.SemaphoreType.DMA,
            pltpu.SemaphoreType.DMA,
        ],
    )(_body)
    return f(table, idxr, lens)


def kernel(item_universe, indices, lengths):
    out, mask = _sampler(item_universe, indices.astype(jnp.int32),
                         lengths.astype(jnp.int32))
    return out, mask
